# baseline (speedup 1.0000x reference)
import jax
import jax.numpy as jnp
from jax import lax
from jax.experimental import pallas as pl
from jax.experimental.pallas import tpu as pltpu
from jax.experimental.pallas import tpu_sc as plsc

_EMB = 64
_SEQ = 200
_NTAGS_PAD = 1024
_NBUF = 6


def _cbow_body(words_hbm, tablet_hbm, wt_hbm, bias_hbm, out_hbm,
               words_v, tiles_v, part_v, shared_sp, sblk_v, sv_v,
               wv, bias_v, res_v, tsems, wsem, bsem):
    c = lax.axis_index("c")
    s = lax.axis_index("s")
    iota16 = lax.iota(jnp.int32, 16)

    wcol0 = pl.multiple_of((s >> 1) * 128, 128)
    wcp = pltpu.async_copy(wt_hbm.at[:, pl.ds(wcol0, 128)], wv, wsem)
    bcp = pltpu.async_copy(bias_hbm.at[pl.ds(s * 64, 64)], bias_v, bsem)

    pltpu.sync_copy(words_hbm, words_v.at[pl.ds(0, _SEQ)])
    wvec = plsc.load_gather(
        words_v, [jnp.minimum(100 * c + s + 16 * iota16, _SEQ - 1)])

    def issue(j):
        w = wvec[j]
        col0 = pl.multiple_of((w >> 7) * 128, 128)
        return pltpu.async_copy(tablet_hbm.at[:, pl.ds(col0, 128)],
                                tiles_v.at[j % _NBUF], tsems[j % _NBUF])

    def extract(j, accs):
        lane = jnp.full((16,), wvec[j] & 127, jnp.int32)
        buf = jnp.full((16,), j % _NBUF, jnp.int32)
        return [acc + plsc.load_gather(tiles_v, [buf, iota16 + g * 16, lane])
                for g, acc in enumerate(accs)]

    accs = [jnp.zeros((16,), jnp.float32) for _ in range(4)]
    cps = [issue(j) for j in range(6)]
    for j in range(6):
        cps[j].wait()
        accs = extract(j, accs)
    for g in range(4):
        part_v[pl.ds(g * 16, 16)] = accs[g]

    @pl.when(s < 4)
    def _seventh():
        cp = issue(6)
        cp.wait()
        accs2 = [part_v[pl.ds(g * 16, 16)] for g in range(4)]
        accs2 = extract(6, accs2)
        for g in range(4):
            part_v[pl.ds(g * 16, 16)] = accs2[g]

    pltpu.sync_copy(part_v, shared_sp.at[s])
    plsc.subcore_barrier()
    pltpu.sync_copy(shared_sp, sblk_v)

    def red_body(r, accs):
        row = jnp.full((16,), r, jnp.int32)
        return tuple(
            acc + plsc.load_gather(sblk_v, [row, iota16 + g * 16])
            for g, acc in enumerate(accs))

    svecs = lax.fori_loop(
        0, 16, red_body, tuple(jnp.zeros((16,), jnp.float32) for _ in range(4)))
    for g in range(4):
        sv_v[pl.ds(g * 16, 16)] = svecs[g]

    wcp.wait()
    bcp.wait()
    lanes0 = (s & 1) * 64 + iota16

    def mv_body(e, accs):
        erow = jnp.full((16,), e, jnp.int32)
        se16 = plsc.load_gather(sv_v, [erow])
        return tuple(
            acc + se16 * plsc.load_gather(wv, [erow, lanes0 + q * 16])
            for q, acc in enumerate(accs))

    zero16 = jnp.zeros((16,), jnp.float32)
    accs_o = lax.fori_loop(
        0, _EMB, mv_body,
        tuple(jnp.where(c == 0, bias_v[pl.ds(q * 16, 16)], zero16)
              for q in range(4)))
    for q in range(4):
        res_v[pl.ds(q * 16, 16)] = accs_o[q]
    pltpu.sync_copy(res_v, out_hbm.at[pl.ds(c * 1024 + s * 64, 64)])


_mesh = plsc.VectorSubcoreMesh(core_axis_name="c", subcore_axis_name="s",
                               num_cores=2, num_subcores=16)

_cbow_call = pl.kernel(
    _cbow_body,
    out_type=jax.ShapeDtypeStruct((2 * _NTAGS_PAD,), jnp.float32),
    mesh=_mesh,
    scratch_types=[
        pltpu.VMEM((256,), jnp.int32),
        pltpu.VMEM((_NBUF, _EMB, 128), jnp.float32),
        pltpu.VMEM((128,), jnp.float32),
        pltpu.VMEM_SHARED((16, 128), jnp.float32),
        pltpu.VMEM((16, 128), jnp.float32),
        pltpu.VMEM((_EMB,), jnp.float32),
        pltpu.VMEM((_EMB, 128), jnp.float32),
        pltpu.VMEM((64,), jnp.float32),
        pltpu.VMEM((64,), jnp.float32),
        [pltpu.SemaphoreType.DMA] * _NBUF,
        pltpu.SemaphoreType.DMA,
        pltpu.SemaphoreType.DMA,
    ],
    compiler_params=pltpu.CompilerParams(use_tc_tiling_on_sc=True,
                                         needs_layout_passes=False),
)


@jax.jit
def kernel(words, emb_table, W, bias):
    words = words.astype(jnp.int32)
    out = _cbow_call(words, emb_table.T, W.T, bias.reshape(-1))
    out = out[:_NTAGS_PAD] + out[_NTAGS_PAD:]
    return out[: bias.size].reshape(1, -1)


# speedup vs baseline: 1.0339x; 1.0339x over previous
import jax
import jax.numpy as jnp
from jax import lax
from jax.experimental import pallas as pl
from jax.experimental.pallas import tpu as pltpu
from jax.experimental.pallas import tpu_sc as plsc

_EMB = 64
_SEQ = 200
_NTAGS_PAD = 1024
_NBUF = 6


def _cbow_body(words_hbm, tablet_hbm, wt_hbm, bias_hbm, out_hbm,
               words_v, tiles_v, part_v, shared_sp, sblk_v, sv_v,
               wv, bias_v, res_v, tsems, wsem, bsem):
    s = lax.axis_index("s")
    iota16 = lax.iota(jnp.int32, 16)

    wcol0 = pl.multiple_of((s >> 1) * 128, 128)
    wcp = pltpu.async_copy(wt_hbm.at[:, pl.ds(wcol0, 128)], wv, wsem)
    bcp = pltpu.async_copy(bias_hbm.at[pl.ds(s * 64, 64)], bias_v, bsem)

    pltpu.sync_copy(words_hbm, words_v.at[pl.ds(0, _SEQ)])
    wvec = plsc.load_gather(words_v, [s + 16 * iota16])

    def issue(j):
        w = wvec[j]
        col0 = pl.multiple_of((w >> 7) * 128, 128)
        return pltpu.async_copy(tablet_hbm.at[:, pl.ds(col0, 128)],
                                tiles_v.at[j % _NBUF], tsems[j % _NBUF])

    def extract(j, accs):
        lane = jnp.full((16,), wvec[j] & 127, jnp.int32)
        buf = jnp.full((16,), j % _NBUF, jnp.int32)
        return [acc + plsc.load_gather(tiles_v, [buf, iota16 + g * 16, lane])
                for g, acc in enumerate(accs)]

    accs = [jnp.zeros((16,), jnp.float32) for _ in range(4)]
    cps = [issue(j) for j in range(_NBUF - 1)]
    for j in range(_NBUF - 1, 12):
        cps.append(issue(j))
        cps[j - (_NBUF - 1)].wait()
        accs = extract(j - (_NBUF - 1), accs)
    for j in range(12 - (_NBUF - 1), 12):
        cps[j].wait()
        accs = extract(j, accs)
    for g in range(4):
        part_v[pl.ds(g * 16, 16)] = accs[g]

    @pl.when(s < _SEQ - 192)
    def _thirteenth():
        cp = issue(12)
        cp.wait()
        accs2 = [part_v[pl.ds(g * 16, 16)] for g in range(4)]
        accs2 = extract(12, accs2)
        for g in range(4):
            part_v[pl.ds(g * 16, 16)] = accs2[g]

    pltpu.sync_copy(part_v, shared_sp.at[s])
    plsc.subcore_barrier()
    pltpu.sync_copy(shared_sp, sblk_v)

    def red_body(r, accs):
        row = jnp.full((16,), r, jnp.int32)
        return tuple(
            acc + plsc.load_gather(sblk_v, [row, iota16 + g * 16])
            for g, acc in enumerate(accs))

    svecs = lax.fori_loop(
        0, 16, red_body, tuple(jnp.zeros((16,), jnp.float32) for _ in range(4)))
    for g in range(4):
        sv_v[pl.ds(g * 16, 16)] = svecs[g]

    wcp.wait()
    bcp.wait()
    lanes0 = (s & 1) * 64 + iota16

    def mv_body(e, accs):
        erow = jnp.full((16,), e, jnp.int32)
        se16 = plsc.load_gather(sv_v, [erow])
        return tuple(
            acc + se16 * plsc.load_gather(wv, [erow, lanes0 + q * 16])
            for q, acc in enumerate(accs))

    accs_o = lax.fori_loop(
        0, _EMB, mv_body, tuple(bias_v[pl.ds(q * 16, 16)] for q in range(4)))
    for q in range(4):
        res_v[pl.ds(q * 16, 16)] = accs_o[q]
    pltpu.sync_copy(res_v, out_hbm.at[pl.ds(s * 64, 64)])


_mesh = plsc.VectorSubcoreMesh(core_axis_name="c", subcore_axis_name="s",
                               num_cores=1, num_subcores=16)

_cbow_call = pl.kernel(
    _cbow_body,
    out_type=jax.ShapeDtypeStruct((_NTAGS_PAD,), jnp.float32),
    mesh=_mesh,
    scratch_types=[
        pltpu.VMEM((256,), jnp.int32),
        pltpu.VMEM((_NBUF, _EMB, 128), jnp.float32),
        pltpu.VMEM((128,), jnp.float32),
        pltpu.VMEM_SHARED((16, 128), jnp.float32),
        pltpu.VMEM((16, 128), jnp.float32),
        pltpu.VMEM((_EMB,), jnp.float32),
        pltpu.VMEM((_EMB, 128), jnp.float32),
        pltpu.VMEM((64,), jnp.float32),
        pltpu.VMEM((64,), jnp.float32),
        [pltpu.SemaphoreType.DMA] * _NBUF,
        pltpu.SemaphoreType.DMA,
        pltpu.SemaphoreType.DMA,
    ],
    compiler_params=pltpu.CompilerParams(use_tc_tiling_on_sc=True,
                                         needs_layout_passes=False),
)


@jax.jit
def kernel(words, emb_table, W, bias):
    words = words.astype(jnp.int32)
    out = _cbow_call(words, emb_table.T, W.T, bias.reshape(-1))
    return out[: bias.size].reshape(1, -1)
